# baseline (speedup 1.0000x reference)
import numpy as np

import jax
import jax.numpy as jnp
from jax import lax
from jax.experimental import pallas as pl
from jax.experimental.pallas import tpu as pltpu
from jax.experimental.pallas import tpu_sc as plsc

N = 10000
E = 320000
D = 128
G = 64

NC = 2
NS = 16
LANES = 16
DH = D // NC

CH = 128
SS = 4
SUPERS = 40
CPT = SUPERS * SS
EPT = CPT * CH
E_PAD = EPT * NS
ROWS_E = E_PAD // CH

N_PAD = 10240
RPT = N_PAD // NS
ZR = 128

PERM = np.concatenate(
    [np.concatenate([b * 32 + 2 * np.arange(16),
                     b * 32 + 2 * np.arange(16) + 1])
     for b in range(D // 32)])


def _head_write(h, h_ref, s_ref, d_ref, as_, ad_):
    hb = h.astype(jnp.bfloat16)
    h_ref[0] = hb[:, :DH]
    h_ref[1] = hb[:, DH:]
    s_ref[...] = jnp.sum(h * as_[None, :], axis=1)
    d_ref[...] = jnp.sum(h * ad_[None, :], axis=1)


def _head_body(x_ref, w_ref, as_ref, ad_ref, h_ref, s_ref, d_ref):
    h = jnp.dot(x_ref[...], w_ref[...], preferred_element_type=jnp.float32)
    _head_write(h, h_ref, s_ref, d_ref, as_ref[...], ad_ref[...])


_head_out_shape = [
    jax.ShapeDtypeStruct((NC, N_PAD, DH), jnp.bfloat16),
    jax.ShapeDtypeStruct((N_PAD,), jnp.float32),
    jax.ShapeDtypeStruct((N_PAD,), jnp.float32),
]


def _tc_head(x_pad, W, a_s, a_d):
    return pl.pallas_call(
        _head_body,
        out_shape=_head_out_shape,
    )(x_pad, W, a_s, a_d)


def _normalize(o_ref, z_ref, b_ref):
    zz = z_ref[...] + 1e-16
    xx = jnp.concatenate([o_ref[0], o_ref[1]], axis=1) / zz[:, None]
    return xx + b_ref[...][None, :]


def _norm_head_body(o_ref, z_ref, b_ref, w_ref, as_ref, ad_ref,
                    h_ref, s_ref, d_ref):
    xx = jnp.maximum(_normalize(o_ref, z_ref, b_ref), 0.0)
    rows = lax.broadcasted_iota(jnp.int32, (N_PAD, 1), 0)
    xx = jnp.where(rows < N, xx, 0.0)
    h = jnp.dot(xx, w_ref[...], preferred_element_type=jnp.float32)
    _head_write(h, h_ref, s_ref, d_ref, as_ref[...], ad_ref[...])


def _tc_norm_head(o_p, z_p, b, W, a_s, a_d):
    return pl.pallas_call(
        _norm_head_body,
        out_shape=_head_out_shape,
    )(o_p, z_p, b, W, a_s, a_d)


def _pool_body(o_ref, z_ref, b_ref, batch_ref, lw_ref, lb_ref, out_ref):
    h = _normalize(o_ref, z_ref, b_ref)
    h = h[:N]
    gids = lax.broadcasted_iota(jnp.int32, (G, N), 0)
    onehot = (gids == batch_ref[...][None, :]).astype(jnp.float32)
    sums = jnp.dot(onehot, h, preferred_element_type=jnp.float32)
    cnt = jnp.sum(onehot, axis=1)
    pooled = sums / jnp.maximum(cnt, 1.0)[:, None]
    out = lax.dot_general(pooled, lw_ref[...], (((1,), (1,)), ((), ())),
                          preferred_element_type=jnp.float32)
    out_ref[...] = out + lb_ref[...][None, :]


def _tc_pool(o_p, z_p, b, batch, linW, linb):
    return pl.pallas_call(
        _pool_body,
        out_shape=jax.ShapeDtypeStruct((G, D), jnp.float32),
    )(o_p, z_p, b, batch, linW, linb)


def _edge_body(src_hbm, dst_hbm, as_hbm, ad_hbm, h_hbm,
               out_hbm, z_hbm,
               sidx_v, didx_v, rows_v, rowsf_v, e2_v, as_v, ad_v, zb_v, zv_v,
               sem_i, sem_g, sem_s, sem_z,
               out_sh, z_sh):
    c = lax.axis_index("c")
    s = lax.axis_index("s")

    zero16 = jnp.zeros((LANES,), jnp.float32)

    def _zrow(i, _):
        for j in range(DH // LANES):
            zb_v[i, pl.ds(j * LANES, LANES)] = zero16
        return 0
    lax.fori_loop(0, ZR, _zrow, 0)

    def _zv(i, _):
        zv_v[pl.ds(i * LANES, LANES)] = zero16
        return 0
    lax.fori_loop(0, RPT // LANES, _zv, 0)

    rbase = s * RPT
    for t in range(RPT // ZR):
        pltpu.sync_copy(zb_v, out_sh.at[pl.ds(rbase + t * ZR, ZR)])

    @pl.when(c == 0)
    def _():
        pltpu.sync_copy(zv_v, z_sh.at[pl.ds(rbase, RPT)])

    pltpu.sync_copy(as_hbm, as_v)
    pltpu.sync_copy(ad_hbm, ad_v)

    plsc.subcore_barrier()

    rowbase = s * CPT

    def _super(t, _):
        roff = rowbase + t * SS
        pltpu.async_copy(src_hbm.at[pl.ds(roff, SS)], sidx_v, sem_i).wait()
        pltpu.async_copy(dst_hbm.at[pl.ds(roff, SS)], didx_v, sem_i).wait()

        gathers = [
            pltpu.async_copy(h_hbm.at[c].at[sidx_v.at[u]], rows_v.at[u],
                             sem_g.at[u])
            for u in range(SS)
        ]

        @plsc.parallel_loop(0, SS * (CH // LANES), unroll=4)
        def _e2(g):
            u = g // (CH // LANES)
            goff = (g % (CH // LANES)) * LANES
            isrc = sidx_v[u, pl.ds(goff, LANES)]
            idst = didx_v[u, pl.ds(goff, LANES)]
            e = plsc.load_gather(as_v, [isrc]) + plsc.load_gather(ad_v, [idst])
            e = jnp.where(e >= 0.0, e, 0.2 * e)
            e2_v[u, pl.ds(goff, LANES)] = jnp.exp(e)

        scatters = []
        for u in range(SS):
            gathers[u].wait()

            @pl.when(c == 0)
            def _(u=u):
                pltpu.async_copy(e2_v.at[u], z_sh.at[didx_v.at[u]],
                                 sem_z.at[u], add=True)

            @plsc.parallel_loop(0, CH, unroll=8)
            def _scale(k, u=u):
                sc = plsc.load_gather(
                    e2_v.at[u], [jnp.full((LANES,), k, jnp.int32)])
                for j in range(DH // (2 * LANES)):
                    v = rows_v[u, k, pl.ds(j * 2 * LANES, 2 * LANES)]
                    a, b = plsc.unpack(v, format=plsc.PackFormat.INTERLEAVED)
                    rowsf_v[u, k, pl.ds(j * 2 * LANES, LANES)] = a * sc
                    rowsf_v[u, k, pl.ds(j * 2 * LANES + LANES, LANES)] = (
                        b * sc)

            scatters.append(
                pltpu.async_copy(rowsf_v.at[u], out_sh.at[didx_v.at[u]],
                                 sem_s.at[u], add=True))

        for sc_h in scatters:
            sc_h.wait()

        @pl.when(c == 0)
        def _():
            for u in range(SS):
                pltpu.make_async_copy(e2_v.at[u], z_sh.at[didx_v.at[u]],
                                      sem_z.at[u]).wait()
        return 0

    lax.fori_loop(0, SUPERS, _super, 0)

    plsc.subcore_barrier()
    pltpu.sync_copy(out_sh.at[pl.ds(rbase, RPT)],
                    out_hbm.at[c, pl.ds(rbase, RPT)])

    @pl.when(c == 0)
    def _():
        pltpu.sync_copy(z_sh.at[pl.ds(rbase, RPT)], z_hbm.at[pl.ds(rbase, RPT)])


_edge_kernel = pl.kernel(
    _edge_body,
    out_type=[
        jax.ShapeDtypeStruct((NC, N_PAD, DH), jnp.float32),
        jax.ShapeDtypeStruct((N_PAD,), jnp.float32),
    ],
    mesh=plsc.VectorSubcoreMesh(core_axis_name="c", subcore_axis_name="s"),
    compiler_params=pltpu.CompilerParams(
        needs_layout_passes=False, use_tc_tiling_on_sc=False),
    scratch_types=[
        pltpu.VMEM((SS, CH), jnp.int32),
        pltpu.VMEM((SS, CH), jnp.int32),
        pltpu.VMEM((SS, CH, DH), jnp.bfloat16),
        pltpu.VMEM((SS, CH, DH), jnp.float32),
        pltpu.VMEM((SS, CH), jnp.float32),
        pltpu.VMEM((N_PAD,), jnp.float32),
        pltpu.VMEM((N_PAD,), jnp.float32),
        pltpu.VMEM((ZR, DH), jnp.float32),
        pltpu.VMEM((RPT,), jnp.float32),
        pltpu.SemaphoreType.DMA,
        pltpu.SemaphoreType.DMA((SS,)),
        pltpu.SemaphoreType.DMA((SS,)),
        pltpu.SemaphoreType.DMA((SS,)),
        pltpu.VMEM_SHARED((N_PAD, DH), jnp.float32),
        pltpu.VMEM_SHARED((N_PAD,), jnp.float32),
    ],
)


def kernel(x, edge_index, batch, W1, a_s1, a_d1, b1, W2, a_s2, a_d2, b2,
           W3, a_s3, a_d3, b3, linW, linb):
    src = jnp.concatenate(
        [edge_index[0], jnp.zeros((E_PAD - E,), jnp.int32)]).reshape(
            ROWS_E, CH)
    dst = jnp.concatenate(
        [edge_index[1], jnp.full((E_PAD - E,), N_PAD - 1, jnp.int32)]
    ).reshape(ROWS_E, CH)
    x_pad = jnp.concatenate(
        [x, jnp.zeros((N_PAD - N, D), jnp.float32)], axis=0)

    h, as_, ad_ = _tc_head(x_pad, W1, a_s1, a_d1)
    o_p, z_p = _edge_kernel(src, dst, as_, ad_, h)

    h, as_, ad_ = _tc_norm_head(o_p, z_p, b1[PERM], W2[PERM, :], a_s2, a_d2)
    o_p, z_p = _edge_kernel(src, dst, as_, ad_, h)

    h, as_, ad_ = _tc_norm_head(o_p, z_p, b2[PERM], W3[PERM, :], a_s3, a_d3)
    o_p, z_p = _edge_kernel(src, dst, as_, ad_, h)

    return _tc_pool(o_p, z_p, b3[PERM], batch, linW[:, PERM], linb)


# speedup vs baseline: 1.1814x; 1.1814x over previous
import numpy as np

import jax
import jax.numpy as jnp
from jax import lax
from jax.experimental import pallas as pl
from jax.experimental.pallas import tpu as pltpu
from jax.experimental.pallas import tpu_sc as plsc

N = 10000
E = 320000
D = 128
G = 64

NC = 2
NS = 16
LANES = 16
DH = D // NC

CH = 128
SS = 4
SUPERS = 40
CPT = SUPERS * SS
EPT = CPT * CH
E_PAD = EPT * NS
ROWS_E = E_PAD // CH

N_PAD = 10240
RPT = N_PAD // NS
ZR = 128

PERM = np.concatenate(
    [np.concatenate([b * 32 + 2 * np.arange(16),
                     b * 32 + 2 * np.arange(16) + 1])
     for b in range(D // 32)])


def _head_write(h, h_ref, s_ref, d_ref, as_, ad_):
    hb = h.astype(jnp.bfloat16)
    h_ref[0] = hb[:, :DH]
    h_ref[1] = hb[:, DH:]
    s_ref[...] = jnp.sum(h * as_[None, :], axis=1)
    d_ref[...] = jnp.sum(h * ad_[None, :], axis=1)


def _head_body(x_ref, w_ref, as_ref, ad_ref, h_ref, s_ref, d_ref):
    h = jnp.dot(x_ref[...], w_ref[...], preferred_element_type=jnp.float32)
    _head_write(h, h_ref, s_ref, d_ref, as_ref[...], ad_ref[...])


_head_out_shape = [
    jax.ShapeDtypeStruct((NC, N_PAD, DH), jnp.bfloat16),
    jax.ShapeDtypeStruct((N_PAD,), jnp.float32),
    jax.ShapeDtypeStruct((N_PAD,), jnp.float32),
]


def _tc_head(x_pad, W, a_s, a_d):
    return pl.pallas_call(
        _head_body,
        out_shape=_head_out_shape,
    )(x_pad, W, a_s, a_d)


def _normalize(o_ref, z_ref, b_ref):
    zz = z_ref[...] + 1e-16
    xx = jnp.concatenate([o_ref[0], o_ref[1]], axis=1) / zz[:, None]
    return xx + b_ref[...][None, :]


def _norm_head_body(o_ref, z_ref, b_ref, w_ref, as_ref, ad_ref,
                    h_ref, s_ref, d_ref):
    xx = jnp.maximum(_normalize(o_ref, z_ref, b_ref), 0.0)
    rows = lax.broadcasted_iota(jnp.int32, (N_PAD, 1), 0)
    xx = jnp.where(rows < N, xx, 0.0)
    h = jnp.dot(xx, w_ref[...], preferred_element_type=jnp.float32)
    _head_write(h, h_ref, s_ref, d_ref, as_ref[...], ad_ref[...])


def _tc_norm_head(o_p, z_p, b, W, a_s, a_d):
    return pl.pallas_call(
        _norm_head_body,
        out_shape=_head_out_shape,
    )(o_p, z_p, b, W, a_s, a_d)


def _pool_body(o_ref, z_ref, b_ref, batch_ref, lw_ref, lb_ref, out_ref):
    h = _normalize(o_ref, z_ref, b_ref)
    h = h[:N]
    gids = lax.broadcasted_iota(jnp.int32, (G, N), 0)
    onehot = (gids == batch_ref[...][None, :]).astype(jnp.float32)
    sums = jnp.dot(onehot, h, preferred_element_type=jnp.float32)
    cnt = jnp.sum(onehot, axis=1)
    pooled = sums / jnp.maximum(cnt, 1.0)[:, None]
    out = lax.dot_general(pooled, lw_ref[...], (((1,), (1,)), ((), ())),
                          preferred_element_type=jnp.float32)
    out_ref[...] = out + lb_ref[...][None, :]


def _tc_pool(o_p, z_p, b, batch, linW, linb):
    return pl.pallas_call(
        _pool_body,
        out_shape=jax.ShapeDtypeStruct((G, D), jnp.float32),
    )(o_p, z_p, b, batch, linW, linb)


def _edge_body(src_hbm, dst_hbm, as_hbm, ad_hbm, h_hbm,
               out_hbm, z_hbm,
               sidx_v, didx_v, rows_v, rowsf_v, e2_v, as_v, ad_v, zv_v,
               sem_i, sem_g, sem_s, sem_z,
               out_sh, z_sh):
    c = lax.axis_index("c")
    s = lax.axis_index("s")

    zero16 = jnp.zeros((LANES,), jnp.float32)

    def _zrow(i, _):
        for j in range(DH // LANES):
            rowsf_v[0, i, pl.ds(j * LANES, LANES)] = zero16
        return 0
    lax.fori_loop(0, ZR, _zrow, 0)

    def _zv(i, _):
        zv_v[pl.ds(i * LANES, LANES)] = zero16
        return 0
    lax.fori_loop(0, RPT // LANES, _zv, 0)

    rbase = s * RPT
    for t in range(RPT // ZR):
        pltpu.sync_copy(rowsf_v.at[0], out_sh.at[pl.ds(rbase + t * ZR, ZR)])

    @pl.when(c == 0)
    def _():
        pltpu.sync_copy(zv_v, z_sh.at[pl.ds(rbase, RPT)])

    pltpu.sync_copy(as_hbm, as_v)
    pltpu.sync_copy(ad_hbm, ad_v)

    plsc.subcore_barrier()

    rowbase = s * CPT

    def _roff(t):
        return jnp.minimum(rowbase + t * SS, (s + 1) * CPT - SS)

    def _fire_idx(t, p):
        pltpu.async_copy(src_hbm.at[pl.ds(_roff(t), SS)], sidx_v.at[p],
                         sem_i.at[p, 0])
        pltpu.async_copy(dst_hbm.at[pl.ds(_roff(t), SS)], didx_v.at[p],
                         sem_i.at[p, 1])

    def _wait_idx(t, p):
        pltpu.make_async_copy(src_hbm.at[pl.ds(_roff(t), SS)], sidx_v.at[p],
                              sem_i.at[p, 0]).wait()
        pltpu.make_async_copy(dst_hbm.at[pl.ds(_roff(t), SS)], didx_v.at[p],
                              sem_i.at[p, 1]).wait()

    def _drain_prev(q):
        for u in range(SS):
            pltpu.make_async_copy(rowsf_v.at[u], out_sh.at[didx_v.at[q].at[u]],
                                  sem_s.at[q, u]).wait()

        @pl.when(c == 0)
        def _():
            for u in range(SS):
                pltpu.make_async_copy(e2_v.at[q, u],
                                      z_sh.at[didx_v.at[q].at[u]],
                                      sem_z.at[q, u]).wait()

    def _super(T, t, p, first):
        _wait_idx(t, p)

        gathers = [
            pltpu.async_copy(h_hbm.at[c].at[sidx_v.at[p].at[u]], rows_v.at[u],
                             sem_g.at[u])
            for u in range(SS)
        ]

        @plsc.parallel_loop(0, SS * (CH // LANES), unroll=4)
        def _e2(g):
            u = g // (CH // LANES)
            goff = (g % (CH // LANES)) * LANES
            isrc = sidx_v[p, u, pl.ds(goff, LANES)]
            idst = didx_v[p, u, pl.ds(goff, LANES)]
            e = plsc.load_gather(as_v, [isrc]) + plsc.load_gather(ad_v, [idst])
            e = jnp.where(e >= 0.0, e, 0.2 * e)
            e2_v[p, u, pl.ds(goff, LANES)] = jnp.exp(e)

        if first:
            @pl.when(T > 0)
            def _():
                _drain_prev(1 - p)
        else:
            _drain_prev(1 - p)
        _fire_idx(t + 1, 1 - p)

        for u in range(SS):
            gathers[u].wait()

            @pl.when(c == 0)
            def _(u=u):
                pltpu.async_copy(e2_v.at[p, u], z_sh.at[didx_v.at[p].at[u]],
                                 sem_z.at[p, u], add=True)

            @plsc.parallel_loop(0, CH, unroll=8)
            def _scale(k, u=u):
                sc = plsc.load_gather(
                    e2_v.at[p, u], [jnp.full((LANES,), k, jnp.int32)])
                for j in range(DH // (2 * LANES)):
                    v = rows_v[u, k, pl.ds(j * 2 * LANES, 2 * LANES)]
                    a, b = plsc.unpack(v, format=plsc.PackFormat.INTERLEAVED)
                    rowsf_v[u, k, pl.ds(j * 2 * LANES, LANES)] = a * sc
                    rowsf_v[u, k, pl.ds(j * 2 * LANES + LANES, LANES)] = (
                        b * sc)

            pltpu.async_copy(rowsf_v.at[u], out_sh.at[didx_v.at[p].at[u]],
                             sem_s.at[p, u], add=True)

    def _pair(T, _):
        _super(T, 2 * T, 0, True)
        _super(T, 2 * T + 1, 1, False)
        return 0

    _fire_idx(0, 0)
    lax.fori_loop(0, SUPERS // 2, _pair, 0)

    _drain_prev(1)
    _wait_idx(SUPERS, 0)

    plsc.subcore_barrier()
    pltpu.sync_copy(out_sh.at[pl.ds(rbase, RPT)],
                    out_hbm.at[c, pl.ds(rbase, RPT)])

    @pl.when(c == 0)
    def _():
        pltpu.sync_copy(z_sh.at[pl.ds(rbase, RPT)], z_hbm.at[pl.ds(rbase, RPT)])


_edge_kernel = pl.kernel(
    _edge_body,
    out_type=[
        jax.ShapeDtypeStruct((NC, N_PAD, DH), jnp.float32),
        jax.ShapeDtypeStruct((N_PAD,), jnp.float32),
    ],
    mesh=plsc.VectorSubcoreMesh(core_axis_name="c", subcore_axis_name="s"),
    compiler_params=pltpu.CompilerParams(
        needs_layout_passes=False, use_tc_tiling_on_sc=False),
    scratch_types=[
        pltpu.VMEM((2, SS, CH), jnp.int32),
        pltpu.VMEM((2, SS, CH), jnp.int32),
        pltpu.VMEM((SS, CH, DH), jnp.bfloat16),
        pltpu.VMEM((SS, CH, DH), jnp.float32),
        pltpu.VMEM((2, SS, CH), jnp.float32),
        pltpu.VMEM((N_PAD,), jnp.float32),
        pltpu.VMEM((N_PAD,), jnp.float32),
        pltpu.VMEM((RPT,), jnp.float32),
        pltpu.SemaphoreType.DMA((2, 2)),
        pltpu.SemaphoreType.DMA((SS,)),
        pltpu.SemaphoreType.DMA((2, SS)),
        pltpu.SemaphoreType.DMA((2, SS)),
        pltpu.VMEM_SHARED((N_PAD, DH), jnp.float32),
        pltpu.VMEM_SHARED((N_PAD,), jnp.float32),
    ],
)


def kernel(x, edge_index, batch, W1, a_s1, a_d1, b1, W2, a_s2, a_d2, b2,
           W3, a_s3, a_d3, b3, linW, linb):
    src = jnp.concatenate(
        [edge_index[0], jnp.zeros((E_PAD - E,), jnp.int32)]).reshape(
            ROWS_E, CH)
    dst = jnp.concatenate(
        [edge_index[1], jnp.full((E_PAD - E,), N_PAD - 1, jnp.int32)]
    ).reshape(ROWS_E, CH)
    x_pad = jnp.concatenate(
        [x, jnp.zeros((N_PAD - N, D), jnp.float32)], axis=0)

    h, as_, ad_ = _tc_head(x_pad, W1, a_s1, a_d1)
    o_p, z_p = _edge_kernel(src, dst, as_, ad_, h)

    h, as_, ad_ = _tc_norm_head(o_p, z_p, b1[PERM], W2[PERM, :], a_s2, a_d2)
    o_p, z_p = _edge_kernel(src, dst, as_, ad_, h)

    h, as_, ad_ = _tc_norm_head(o_p, z_p, b2[PERM], W3[PERM, :], a_s3, a_d3)
    o_p, z_p = _edge_kernel(src, dst, as_, ad_, h)

    return _tc_pool(o_p, z_p, b3[PERM], batch, linW[:, PERM], linb)


# speedup vs baseline: 1.2994x; 1.0999x over previous
import numpy as np

import jax
import jax.numpy as jnp
from jax import lax
from jax.experimental import pallas as pl
from jax.experimental.pallas import tpu as pltpu
from jax.experimental.pallas import tpu_sc as plsc

N = 10000
E = 320000
D = 128
G = 64

NC = 2
NS = 16
LANES = 16
DH = D // NC

CH = 128
SS = 4
SUPERS = 40
CPT = SUPERS * SS
EPT = CPT * CH
E_PAD = EPT * NS
ROWS_E = E_PAD // CH

N_PAD = 10240
RPT = N_PAD // NS
ZR = 128

PERM = np.concatenate(
    [np.concatenate([b * 32 + 2 * np.arange(16),
                     b * 32 + 2 * np.arange(16) + 1])
     for b in range(D // 32)])


def _head_write(h, h_ref, s_ref, d_ref, as_, ad_):
    hb = h.astype(jnp.bfloat16)
    h_ref[0] = hb[:, :DH]
    h_ref[1] = hb[:, DH:]
    s_ref[...] = jnp.sum(h * as_[None, :], axis=1)
    d_ref[...] = jnp.sum(h * ad_[None, :], axis=1)


def _head_body(x_ref, w_ref, as_ref, ad_ref, h_ref, s_ref, d_ref):
    h = jnp.dot(x_ref[...], w_ref[...], preferred_element_type=jnp.float32)
    _head_write(h, h_ref, s_ref, d_ref, as_ref[...], ad_ref[...])


_head_out_shape = [
    jax.ShapeDtypeStruct((NC, N_PAD, DH), jnp.bfloat16),
    jax.ShapeDtypeStruct((N_PAD,), jnp.float32),
    jax.ShapeDtypeStruct((N_PAD,), jnp.float32),
]


def _tc_head(x_pad, W, a_s, a_d):
    return pl.pallas_call(
        _head_body,
        out_shape=_head_out_shape,
    )(x_pad, W, a_s, a_d)


def _normalize(o_ref, z_ref, b_ref):
    zz = z_ref[...] + 1e-16
    oo = jnp.concatenate([o_ref[0], o_ref[1]], axis=1).astype(jnp.float32)
    xx = oo / zz[:, None]
    return xx + b_ref[...][None, :]


def _norm_head_body(o_ref, z_ref, b_ref, w_ref, as_ref, ad_ref,
                    h_ref, s_ref, d_ref):
    xx = jnp.maximum(_normalize(o_ref, z_ref, b_ref), 0.0)
    rows = lax.broadcasted_iota(jnp.int32, (N_PAD, 1), 0)
    xx = jnp.where(rows < N, xx, 0.0)
    h = jnp.dot(xx, w_ref[...], preferred_element_type=jnp.float32)
    _head_write(h, h_ref, s_ref, d_ref, as_ref[...], ad_ref[...])


def _tc_norm_head(o_p, z_p, b, W, a_s, a_d):
    return pl.pallas_call(
        _norm_head_body,
        out_shape=_head_out_shape,
    )(o_p, z_p, b, W, a_s, a_d)


def _pool_body(o_ref, z_ref, b_ref, batch_ref, lw_ref, lb_ref, out_ref):
    h = _normalize(o_ref, z_ref, b_ref)
    h = h[:N]
    gids = lax.broadcasted_iota(jnp.int32, (G, N), 0)
    onehot = (gids == batch_ref[...][None, :]).astype(jnp.float32)
    sums = jnp.dot(onehot, h, preferred_element_type=jnp.float32)
    cnt = jnp.sum(onehot, axis=1)
    pooled = sums / jnp.maximum(cnt, 1.0)[:, None]
    out = lax.dot_general(pooled, lw_ref[...], (((1,), (1,)), ((), ())),
                          preferred_element_type=jnp.float32)
    out_ref[...] = out + lb_ref[...][None, :]


def _tc_pool(o_p, z_p, b, batch, linW, linb):
    return pl.pallas_call(
        _pool_body,
        out_shape=jax.ShapeDtypeStruct((G, D), jnp.float32),
    )(o_p, z_p, b, batch, linW, linb)


def _edge_body(src_hbm, dst_hbm, as_hbm, ad_hbm, h_hbm,
               out_hbm, z_hbm,
               sidx_v, didx_v, rows_v, rowsf_v, e2_v, as_v, ad_v, zv_v,
               sem_i, sem_g, sem_s, sem_z,
               out_sh, z_sh):
    c = lax.axis_index("c")
    s = lax.axis_index("s")

    zero16 = jnp.zeros((LANES,), jnp.float32)

    zero32b = jnp.zeros((2 * LANES,), jnp.bfloat16)

    def _zrow(i, _):
        for j in range(DH // (2 * LANES)):
            rowsf_v[0, i, pl.ds(j * 2 * LANES, 2 * LANES)] = zero32b
        return 0
    lax.fori_loop(0, ZR, _zrow, 0)

    def _zv(i, _):
        zv_v[pl.ds(i * LANES, LANES)] = zero16
        return 0
    lax.fori_loop(0, RPT // LANES, _zv, 0)

    rbase = s * RPT
    for t in range(RPT // ZR):
        pltpu.sync_copy(rowsf_v.at[0], out_sh.at[pl.ds(rbase + t * ZR, ZR)])

    @pl.when(c == 0)
    def _():
        pltpu.sync_copy(zv_v, z_sh.at[pl.ds(rbase, RPT)])

    pltpu.sync_copy(as_hbm, as_v)
    pltpu.sync_copy(ad_hbm, ad_v)

    plsc.subcore_barrier()

    rowbase = s * CPT

    def _roff(t):
        return jnp.minimum(rowbase + t * SS, (s + 1) * CPT - SS)

    def _fire_idx(t, p):
        pltpu.async_copy(src_hbm.at[pl.ds(_roff(t), SS)], sidx_v.at[p],
                         sem_i.at[p, 0])
        pltpu.async_copy(dst_hbm.at[pl.ds(_roff(t), SS)], didx_v.at[p],
                         sem_i.at[p, 1])

    def _wait_idx(t, p):
        pltpu.make_async_copy(src_hbm.at[pl.ds(_roff(t), SS)], sidx_v.at[p],
                              sem_i.at[p, 0]).wait()
        pltpu.make_async_copy(dst_hbm.at[pl.ds(_roff(t), SS)], didx_v.at[p],
                              sem_i.at[p, 1]).wait()

    def _drain_prev(q):
        for u in range(SS):
            pltpu.make_async_copy(rowsf_v.at[u], out_sh.at[didx_v.at[q].at[u]],
                                  sem_s.at[q, u]).wait()

        @pl.when(c == 0)
        def _():
            for u in range(SS):
                pltpu.make_async_copy(e2_v.at[q, u],
                                      z_sh.at[didx_v.at[q].at[u]],
                                      sem_z.at[q, u]).wait()

    def _super(T, t, p, first):
        _wait_idx(t, p)

        gathers = [
            pltpu.async_copy(h_hbm.at[c].at[sidx_v.at[p].at[u]], rows_v.at[u],
                             sem_g.at[u])
            for u in range(SS)
        ]

        @plsc.parallel_loop(0, SS * (CH // LANES), unroll=4)
        def _e2(g):
            u = g // (CH // LANES)
            goff = (g % (CH // LANES)) * LANES
            isrc = sidx_v[p, u, pl.ds(goff, LANES)]
            idst = didx_v[p, u, pl.ds(goff, LANES)]
            e = plsc.load_gather(as_v, [isrc]) + plsc.load_gather(ad_v, [idst])
            e = jnp.where(e >= 0.0, e, 0.2 * e)
            e2_v[p, u, pl.ds(goff, LANES)] = jnp.exp(e)

        if first:
            @pl.when(T > 0)
            def _():
                _drain_prev(1 - p)
        else:
            _drain_prev(1 - p)
        _fire_idx(t + 1, 1 - p)

        for u in range(SS):
            gathers[u].wait()

            @pl.when(c == 0)
            def _(u=u):
                pltpu.async_copy(e2_v.at[p, u], z_sh.at[didx_v.at[p].at[u]],
                                 sem_z.at[p, u], add=True)

            @plsc.parallel_loop(0, CH, unroll=8)
            def _scale(k, u=u):
                sc = plsc.load_gather(
                    e2_v.at[p, u], [jnp.full((LANES,), k, jnp.int32)])
                for j in range(DH // (2 * LANES)):
                    v = rows_v[u, k, pl.ds(j * 2 * LANES, 2 * LANES)]
                    a, b = plsc.unpack(v, format=plsc.PackFormat.INTERLEAVED)
                    rowsf_v[u, k, pl.ds(j * 2 * LANES, 2 * LANES)] = plsc.pack(
                        a * sc, b * sc, format=plsc.PackFormat.INTERLEAVED)

            pltpu.async_copy(rowsf_v.at[u], out_sh.at[didx_v.at[p].at[u]],
                             sem_s.at[p, u], add=True)

    def _pair(T, _):
        _super(T, 2 * T, 0, True)
        _super(T, 2 * T + 1, 1, False)
        return 0

    _fire_idx(0, 0)
    lax.fori_loop(0, SUPERS // 2, _pair, 0)

    _drain_prev(1)
    _wait_idx(SUPERS, 0)

    plsc.subcore_barrier()
    pltpu.sync_copy(out_sh.at[pl.ds(rbase, RPT)],
                    out_hbm.at[c, pl.ds(rbase, RPT)])

    @pl.when(c == 0)
    def _():
        pltpu.sync_copy(z_sh.at[pl.ds(rbase, RPT)], z_hbm.at[pl.ds(rbase, RPT)])


_edge_kernel = pl.kernel(
    _edge_body,
    out_type=[
        jax.ShapeDtypeStruct((NC, N_PAD, DH), jnp.bfloat16),
        jax.ShapeDtypeStruct((N_PAD,), jnp.float32),
    ],
    mesh=plsc.VectorSubcoreMesh(core_axis_name="c", subcore_axis_name="s"),
    compiler_params=pltpu.CompilerParams(
        needs_layout_passes=False, use_tc_tiling_on_sc=False),
    scratch_types=[
        pltpu.VMEM((2, SS, CH), jnp.int32),
        pltpu.VMEM((2, SS, CH), jnp.int32),
        pltpu.VMEM((SS, CH, DH), jnp.bfloat16),
        pltpu.VMEM((SS, CH, DH), jnp.bfloat16),
        pltpu.VMEM((2, SS, CH), jnp.float32),
        pltpu.VMEM((N_PAD,), jnp.float32),
        pltpu.VMEM((N_PAD,), jnp.float32),
        pltpu.VMEM((RPT,), jnp.float32),
        pltpu.SemaphoreType.DMA((2, 2)),
        pltpu.SemaphoreType.DMA((SS,)),
        pltpu.SemaphoreType.DMA((2, SS)),
        pltpu.SemaphoreType.DMA((2, SS)),
        pltpu.VMEM_SHARED((N_PAD, DH), jnp.bfloat16),
        pltpu.VMEM_SHARED((N_PAD,), jnp.float32),
    ],
)


def kernel(x, edge_index, batch, W1, a_s1, a_d1, b1, W2, a_s2, a_d2, b2,
           W3, a_s3, a_d3, b3, linW, linb):
    src = jnp.concatenate(
        [edge_index[0], jnp.zeros((E_PAD - E,), jnp.int32)]).reshape(
            ROWS_E, CH)
    dst = jnp.concatenate(
        [edge_index[1], jnp.full((E_PAD - E,), N_PAD - 1, jnp.int32)]
    ).reshape(ROWS_E, CH)
    x_pad = jnp.concatenate(
        [x, jnp.zeros((N_PAD - N, D), jnp.float32)], axis=0)

    h, as_, ad_ = _tc_head(x_pad, W1, a_s1, a_d1)
    o_p, z_p = _edge_kernel(src, dst, as_, ad_, h)

    h, as_, ad_ = _tc_norm_head(o_p, z_p, b1, W2, a_s2, a_d2)
    o_p, z_p = _edge_kernel(src, dst, as_, ad_, h)

    h, as_, ad_ = _tc_norm_head(o_p, z_p, b2, W3, a_s3, a_d3)
    o_p, z_p = _edge_kernel(src, dst, as_, ad_, h)

    return _tc_pool(o_p, z_p, b3, batch, linW, linb)


# speedup vs baseline: 1.3879x; 1.0681x over previous
import numpy as np

import jax
import jax.numpy as jnp
from jax import lax
from jax.experimental import pallas as pl
from jax.experimental.pallas import tpu as pltpu
from jax.experimental.pallas import tpu_sc as plsc

N = 10000
E = 320000
D = 128
G = 64

NC = 2
NS = 16
LANES = 16
DH = D // NC

CH = 128
SS = 5
SUPERS = 32
CPT = SUPERS * SS
EPT = CPT * CH
E_PAD = EPT * NS
ROWS_E = E_PAD // CH

N_PAD = 10240
RPT = N_PAD // NS
ZR = 128

PERM = np.concatenate(
    [np.concatenate([b * 32 + 2 * np.arange(16),
                     b * 32 + 2 * np.arange(16) + 1])
     for b in range(D // 32)])


def _head_write(h, h_ref, s_ref, d_ref, as_, ad_):
    hb = h.astype(jnp.bfloat16)
    h_ref[0] = hb[:, :DH]
    h_ref[1] = hb[:, DH:]
    s_ref[...] = jnp.sum(h * as_[None, :], axis=1)
    d_ref[...] = jnp.sum(h * ad_[None, :], axis=1)


def _head_body(x_ref, w_ref, as_ref, ad_ref, h_ref, s_ref, d_ref):
    h = jnp.dot(x_ref[...], w_ref[...], preferred_element_type=jnp.float32)
    _head_write(h, h_ref, s_ref, d_ref, as_ref[...], ad_ref[...])


_head_out_shape = [
    jax.ShapeDtypeStruct((NC, N_PAD, DH), jnp.bfloat16),
    jax.ShapeDtypeStruct((N_PAD,), jnp.float32),
    jax.ShapeDtypeStruct((N_PAD,), jnp.float32),
]


def _tc_head(x_pad, W, a_s, a_d):
    return pl.pallas_call(
        _head_body,
        out_shape=_head_out_shape,
    )(x_pad, W, a_s, a_d)


def _normalize(o_ref, z_ref, b_ref):
    zz = z_ref[...] + 1e-16
    oo = jnp.concatenate([o_ref[0], o_ref[1]], axis=1).astype(jnp.float32)
    xx = oo / zz[:, None]
    return xx + b_ref[...][None, :]


def _norm_head_body(o_ref, z_ref, b_ref, w_ref, as_ref, ad_ref,
                    h_ref, s_ref, d_ref):
    xx = jnp.maximum(_normalize(o_ref, z_ref, b_ref), 0.0)
    rows = lax.broadcasted_iota(jnp.int32, (N_PAD, 1), 0)
    xx = jnp.where(rows < N, xx, 0.0)
    h = jnp.dot(xx, w_ref[...], preferred_element_type=jnp.float32)
    _head_write(h, h_ref, s_ref, d_ref, as_ref[...], ad_ref[...])


def _tc_norm_head(o_p, z_p, b, W, a_s, a_d):
    return pl.pallas_call(
        _norm_head_body,
        out_shape=_head_out_shape,
    )(o_p, z_p, b, W, a_s, a_d)


def _pool_body(o_ref, z_ref, b_ref, batch_ref, lw_ref, lb_ref, out_ref):
    h = _normalize(o_ref, z_ref, b_ref)
    h = h[:N]
    gids = lax.broadcasted_iota(jnp.int32, (G, N), 0)
    onehot = (gids == batch_ref[...][None, :]).astype(jnp.float32)
    sums = jnp.dot(onehot, h, preferred_element_type=jnp.float32)
    cnt = jnp.sum(onehot, axis=1)
    pooled = sums / jnp.maximum(cnt, 1.0)[:, None]
    out = lax.dot_general(pooled, lw_ref[...], (((1,), (1,)), ((), ())),
                          preferred_element_type=jnp.float32)
    out_ref[...] = out + lb_ref[...][None, :]


def _tc_pool(o_p, z_p, b, batch, linW, linb):
    return pl.pallas_call(
        _pool_body,
        out_shape=jax.ShapeDtypeStruct((G, D), jnp.float32),
    )(o_p, z_p, b, batch, linW, linb)


def _edge_body(src_hbm, dst_hbm, as_hbm, ad_hbm, h_hbm,
               out_hbm, z_hbm,
               sidx_v, didx_v, rows_v, rowsf_v, e2_v, as_v, ad_v, zv_v,
               sem_i, sem_g, sem_s, sem_z,
               out_sh, z_sh):
    c = lax.axis_index("c")
    s = lax.axis_index("s")

    zero16 = jnp.zeros((LANES,), jnp.float32)

    zero32b = jnp.zeros((2 * LANES,), jnp.bfloat16)

    def _zrow(i, _):
        for j in range(DH // (2 * LANES)):
            rowsf_v[0, i, pl.ds(j * 2 * LANES, 2 * LANES)] = zero32b
        return 0
    lax.fori_loop(0, ZR, _zrow, 0)

    def _zv(i, _):
        zv_v[pl.ds(i * LANES, LANES)] = zero16
        return 0
    lax.fori_loop(0, RPT // LANES, _zv, 0)

    rbase = s * RPT
    for t in range(RPT // ZR):
        pltpu.sync_copy(rowsf_v.at[0], out_sh.at[pl.ds(rbase + t * ZR, ZR)])

    @pl.when(c == 0)
    def _():
        pltpu.sync_copy(zv_v, z_sh.at[pl.ds(rbase, RPT)])

    pltpu.sync_copy(as_hbm, as_v)
    pltpu.sync_copy(ad_hbm, ad_v)

    plsc.subcore_barrier()

    rowbase = s * CPT

    def _roff(t):
        return jnp.minimum(rowbase + t * SS, (s + 1) * CPT - SS)

    def _fire_idx(t, p):
        pltpu.async_copy(src_hbm.at[pl.ds(_roff(t), SS)], sidx_v.at[p],
                         sem_i.at[p, 0])
        pltpu.async_copy(dst_hbm.at[pl.ds(_roff(t), SS)], didx_v.at[p],
                         sem_i.at[p, 1])

    def _wait_idx(t, p):
        pltpu.make_async_copy(src_hbm.at[pl.ds(_roff(t), SS)], sidx_v.at[p],
                              sem_i.at[p, 0]).wait()
        pltpu.make_async_copy(dst_hbm.at[pl.ds(_roff(t), SS)], didx_v.at[p],
                              sem_i.at[p, 1]).wait()

    def _drain_prev(q):
        for u in range(SS):
            pltpu.make_async_copy(rowsf_v.at[u], out_sh.at[didx_v.at[q].at[u]],
                                  sem_s.at[q, u]).wait()

        @pl.when(c == 0)
        def _():
            for u in range(SS):
                pltpu.make_async_copy(e2_v.at[q, u],
                                      z_sh.at[didx_v.at[q].at[u]],
                                      sem_z.at[q, u]).wait()

    def _super(T, t, p, first):
        _wait_idx(t, p)

        gathers = [
            pltpu.async_copy(h_hbm.at[c].at[sidx_v.at[p].at[u]], rows_v.at[u],
                             sem_g.at[u])
            for u in range(SS)
        ]

        @plsc.parallel_loop(0, SS * (CH // LANES), unroll=4)
        def _e2(g):
            u = g // (CH // LANES)
            goff = (g % (CH // LANES)) * LANES
            isrc = sidx_v[p, u, pl.ds(goff, LANES)]
            idst = didx_v[p, u, pl.ds(goff, LANES)]
            e = plsc.load_gather(as_v, [isrc]) + plsc.load_gather(ad_v, [idst])
            e = jnp.where(e >= 0.0, e, 0.2 * e)
            e2_v[p, u, pl.ds(goff, LANES)] = jnp.exp(e)

        if first:
            @pl.when(T > 0)
            def _():
                _drain_prev(1 - p)
        else:
            _drain_prev(1 - p)
        _fire_idx(t + 1, 1 - p)

        for u in range(SS):
            gathers[u].wait()

            @pl.when(c == 0)
            def _(u=u):
                pltpu.async_copy(e2_v.at[p, u], z_sh.at[didx_v.at[p].at[u]],
                                 sem_z.at[p, u], add=True)

            @plsc.parallel_loop(0, CH, unroll=8)
            def _scale(k, u=u):
                sc = plsc.load_gather(
                    e2_v.at[p, u], [jnp.full((LANES,), k, jnp.int32)])
                for j in range(DH // (2 * LANES)):
                    v = rows_v[u, k, pl.ds(j * 2 * LANES, 2 * LANES)]
                    a, b = plsc.unpack(v, format=plsc.PackFormat.INTERLEAVED)
                    rowsf_v[u, k, pl.ds(j * 2 * LANES, 2 * LANES)] = plsc.pack(
                        a * sc, b * sc, format=plsc.PackFormat.INTERLEAVED)

            pltpu.async_copy(rowsf_v.at[u], out_sh.at[didx_v.at[p].at[u]],
                             sem_s.at[p, u], add=True)

    def _pair(T, _):
        _super(T, 2 * T, 0, True)
        _super(T, 2 * T + 1, 1, False)
        return 0

    _fire_idx(0, 0)
    lax.fori_loop(0, SUPERS // 2, _pair, 0)

    _drain_prev(1)
    _wait_idx(SUPERS, 0)

    plsc.subcore_barrier()
    pltpu.sync_copy(out_sh.at[pl.ds(rbase, RPT)],
                    out_hbm.at[c, pl.ds(rbase, RPT)])

    @pl.when(c == 0)
    def _():
        pltpu.sync_copy(z_sh.at[pl.ds(rbase, RPT)], z_hbm.at[pl.ds(rbase, RPT)])


_edge_kernel = pl.kernel(
    _edge_body,
    out_type=[
        jax.ShapeDtypeStruct((NC, N_PAD, DH), jnp.bfloat16),
        jax.ShapeDtypeStruct((N_PAD,), jnp.float32),
    ],
    mesh=plsc.VectorSubcoreMesh(core_axis_name="c", subcore_axis_name="s"),
    compiler_params=pltpu.CompilerParams(
        needs_layout_passes=False, use_tc_tiling_on_sc=False),
    scratch_types=[
        pltpu.VMEM((2, SS, CH), jnp.int32),
        pltpu.VMEM((2, SS, CH), jnp.int32),
        pltpu.VMEM((SS, CH, DH), jnp.bfloat16),
        pltpu.VMEM((SS, CH, DH), jnp.bfloat16),
        pltpu.VMEM((2, SS, CH), jnp.float32),
        pltpu.VMEM((N_PAD,), jnp.float32),
        pltpu.VMEM((N_PAD,), jnp.float32),
        pltpu.VMEM((RPT,), jnp.float32),
        pltpu.SemaphoreType.DMA((2, 2)),
        pltpu.SemaphoreType.DMA((SS,)),
        pltpu.SemaphoreType.DMA((2, SS)),
        pltpu.SemaphoreType.DMA((2, SS)),
        pltpu.VMEM_SHARED((N_PAD, DH), jnp.bfloat16),
        pltpu.VMEM_SHARED((N_PAD,), jnp.float32),
    ],
)


def kernel(x, edge_index, batch, W1, a_s1, a_d1, b1, W2, a_s2, a_d2, b2,
           W3, a_s3, a_d3, b3, linW, linb):
    src = jnp.concatenate(
        [edge_index[0], jnp.zeros((E_PAD - E,), jnp.int32)]).reshape(
            ROWS_E, CH)
    dst = jnp.concatenate(
        [edge_index[1], jnp.full((E_PAD - E,), N_PAD - 1, jnp.int32)]
    ).reshape(ROWS_E, CH)
    x_pad = jnp.concatenate(
        [x, jnp.zeros((N_PAD - N, D), jnp.float32)], axis=0)

    h, as_, ad_ = _tc_head(x_pad, W1, a_s1, a_d1)
    o_p, z_p = _edge_kernel(src, dst, as_, ad_, h)

    h, as_, ad_ = _tc_norm_head(o_p, z_p, b1, W2, a_s2, a_d2)
    o_p, z_p = _edge_kernel(src, dst, as_, ad_, h)

    h, as_, ad_ = _tc_norm_head(o_p, z_p, b2, W3, a_s3, a_d3)
    o_p, z_p = _edge_kernel(src, dst, as_, ad_, h)

    return _tc_pool(o_p, z_p, b3, batch, linW, linb)
